# baseline (speedup 1.0000x reference)
import functools

import jax
import jax.numpy as jnp
from jax import lax
from jax.experimental import pallas as pl
from jax.experimental.pallas import tpu as pltpu
from jax.experimental.pallas import tpu_sc as plsc

B = 4096
NUM_FIELDS = 26
NUM_DEEP_FIELDS = 24
VOCAB = 1000
NUM_DENSE = 13
LATENT = 64
D_EMB = NUM_DEEP_FIELDS * LATENT

_BT = 512
_LANES = 16
_NCHUNKS = 2
_CHUNK = B // _NCHUNKS
_BPW = 128

_DIMS_T = (((1,), (1,)), ((), ()))

_SC_PARAMS = pltpu.CompilerParams(needs_layout_passes=False)


def _sc_info():
    info = plsc.get_sparse_core_info()
    return info.num_cores, info.num_subcores


def _sc_deep_fn(chunk_lo):
    nc, ns = _sc_info()
    nwu = _CHUNK // _BPW
    nch = _BPW // _LANES

    mesh = plsc.VectorSubcoreMesh(core_axis_name="c", subcore_axis_name="s")

    @functools.partial(
        pl.kernel,
        mesh=mesh,
        compiler_params=_SC_PARAMS,
        out_type=jax.ShapeDtypeStruct((NUM_DEEP_FIELDS, _CHUNK, 128),
                                      jnp.float32),
        scratch_types=[
            pltpu.VMEM((NUM_DEEP_FIELDS, _BPW), jnp.int32),
            pltpu.VMEM((_BPW,), jnp.int32),
            pltpu.VMEM((_BPW,), jnp.int32),
            pltpu.VMEM((_BPW, 128), jnp.float32),
            pltpu.VMEM((_BPW, 128), jnp.float32),
            pltpu.SemaphoreType.DMA,
            pltpu.SemaphoreType.DMA,
        ],
    )
    def sc_deep(sparse_hbm, emb_hbm, e3_hbm,
                ids_v, idx0_v, idx1_v, rows0_v, rows1_v, gsem, wsem):
        wid = lax.axis_index("s") * nc + lax.axis_index("c")

        @pl.when(wid < nwu)
        def _():
            base = wid * _BPW
            idx_bufs = (idx0_v, idx1_v)
            row_bufs = (rows0_v, rows1_v)

            def build_idx(f):
                buf = idx_bufs[f % 2]
                for c in range(nch):
                    buf[pl.ds(c * _LANES, _LANES)] = (
                        ids_v[f, pl.ds(c * _LANES, _LANES)] + f * VOCAB
                    )

            pltpu.sync_copy(
                sparse_hbm.at[pl.ds(0, NUM_DEEP_FIELDS),
                              pl.ds(chunk_lo + base, _BPW)],
                ids_v,
            )
            build_idx(0)
            gather = pltpu.async_copy(emb_hbm.at[idx0_v], rows0_v, gsem)
            write = None
            for f in range(NUM_DEEP_FIELDS):
                cur = f % 2
                if f + 1 < NUM_DEEP_FIELDS:
                    build_idx(f + 1)
                gather.wait()
                if write is not None:
                    write.wait()
                write = pltpu.async_copy(
                    row_bufs[cur], e3_hbm.at[f, pl.ds(base, _BPW)], wsem
                )
                if f + 1 < NUM_DEEP_FIELDS:
                    gather = pltpu.async_copy(
                        emb_hbm.at[idx_bufs[1 - cur]], row_bufs[1 - cur], gsem
                    )
            write.wait()

    return sc_deep


def _sc_wide_fn():
    nc, ns = _sc_info()
    nw = nc * ns
    bpw = B // nw
    nch = bpw // _LANES

    mesh = plsc.VectorSubcoreMesh(core_axis_name="c", subcore_axis_name="s")

    @functools.partial(
        pl.kernel,
        mesh=mesh,
        compiler_params=_SC_PARAMS,
        out_type=jax.ShapeDtypeStruct((B, 1), jnp.float32),
        scratch_types=[
            pltpu.VMEM((NUM_FIELDS, bpw), jnp.int32),
            pltpu.VMEM((NUM_FIELDS * VOCAB,), jnp.float32),
            pltpu.VMEM((bpw, 1), jnp.float32),
        ],
    )
    def sc_wide(sparse_hbm, wide_sp_hbm, wide_out_hbm, ids_v, wtab_v, wsum_v):
        wid = lax.axis_index("s") * nc + lax.axis_index("c")
        base = wid * bpw
        pltpu.sync_copy(sparse_hbm.at[:, pl.ds(base, bpw)], ids_v)
        pltpu.sync_copy(wide_sp_hbm, wtab_v)
        iota = lax.iota(jnp.int32, _LANES)
        zeros = jnp.zeros((_LANES,), jnp.int32)
        for c in range(nch):
            acc = jnp.zeros((_LANES,), jnp.float32)
            for f in range(NUM_FIELDS):
                ids = ids_v[f, pl.ds(c * _LANES, _LANES)] + f * VOCAB
                acc = acc + plsc.load_gather(wtab_v, [ids])
            plsc.store_scatter(wsum_v, [iota + c * _LANES, zeros], acc)
        pltpu.sync_copy(wsum_v, wide_out_hbm.at[pl.ds(base, bpw)])

    return sc_wide


def _tc_mlp(e3_ref, dense_ref, wsum_ref, dw_ref, db_ref, w1_ref, b1_ref,
            w2_ref, b2_ref, w3_ref, b3_ref, wout_ref, ww13_ref, bias_ref,
            out_ref):
    f32 = jnp.float32
    dot_t = functools.partial(
        lax.dot_general, dimension_numbers=_DIMS_T, preferred_element_type=f32
    )
    dense = dense_ref[...]
    d0 = dot_t(dense, dw_ref[...]) + db_ref[...][None, :]
    hcat = jnp.concatenate(
        [d0] + [e3_ref[f][:, :LATENT] for f in range(NUM_DEEP_FIELDS)], axis=1
    )
    h = jnp.maximum(dot_t(hcat, w1_ref[...]) + b1_ref[...][None, :], 0.0)
    h = jnp.maximum(dot_t(h, w2_ref[...]) + b2_ref[...][None, :], 0.0)
    h = jnp.maximum(dot_t(h, w3_ref[...]) + b3_ref[...][None, :], 0.0)
    deep = jnp.sum(h * wout_ref[...], axis=1, keepdims=True)
    wide_dense = jnp.sum(dense * ww13_ref[...], axis=1, keepdims=True)
    out_ref[...] = deep + wide_dense + wsum_ref[...] + bias_ref[...]


def _tc_call(k, e3c, dense, wsum, dense_w, dense_b, w1, b1, w2, b2, w3, b3,
             w_out, ww13, bias):
    t0 = k * (_CHUNK // _BT)
    grid = (_CHUNK // _BT,)
    full = lambda shape: pl.BlockSpec(shape, lambda i: tuple(0 for _ in shape))
    return pl.pallas_call(
        _tc_mlp,
        grid=grid,
        in_specs=[
            pl.BlockSpec((NUM_DEEP_FIELDS, _BT, 128), lambda i: (0, i, 0)),
            pl.BlockSpec((_BT, NUM_DENSE), lambda i: (t0 + i, 0)),
            pl.BlockSpec((_BT, 1), lambda i: (t0 + i, 0)),
            full((LATENT, NUM_DENSE)),
            pl.BlockSpec((LATENT,), lambda i: (0,)),
            full((1024, LATENT + D_EMB)),
            pl.BlockSpec((1024,), lambda i: (0,)),
            full((512, 1024)),
            pl.BlockSpec((512,), lambda i: (0,)),
            full((256, 512)),
            pl.BlockSpec((256,), lambda i: (0,)),
            full((1, 256)),
            full((1, NUM_DENSE)),
            full((1, 1)),
        ],
        out_specs=pl.BlockSpec((_BT, 1), lambda i: (i, 0)),
        out_shape=jax.ShapeDtypeStruct((_CHUNK, 1), jnp.float32),
    )(e3c, dense, wsum, dense_w, dense_b, w1, b1, w2, b2, w3, b3,
      w_out, ww13, bias)


def kernel(sparse_features, dense_features, wide_w, dense_w, dense_b, emb,
           w1, b1, w2, b2, w3, b3, w_out, bias):
    emb_flat = jnp.pad(
        emb.reshape(NUM_DEEP_FIELDS * VOCAB, LATENT),
        ((0, 0), (0, 128 - LATENT)),
    )
    wide_sp = wide_w[NUM_DENSE:]
    ww13 = wide_w[:NUM_DENSE][None, :]

    wsum = _sc_wide_fn()(sparse_features, wide_sp)

    outs = []
    for k in range(_NCHUNKS):
        e3c = _sc_deep_fn(k * _CHUNK)(sparse_features, emb_flat)
        outs.append(_tc_call(
            k, e3c, dense_features, wsum,
            dense_w, dense_b, w1, b1, w2, b2, w3, b3, w_out, ww13, bias,
        ))
    return jnp.concatenate(outs, axis=0)


# speedup vs baseline: 1.2467x; 1.2467x over previous
import functools

import jax
import jax.numpy as jnp
from jax import lax
from jax.experimental import pallas as pl
from jax.experimental.pallas import tpu as pltpu
from jax.experimental.pallas import tpu_sc as plsc

B = 4096
NUM_FIELDS = 26
NUM_DEEP_FIELDS = 24
VOCAB = 1000
NUM_DENSE = 13
LATENT = 64
D_EMB = NUM_DEEP_FIELDS * LATENT

_BT = 1024
_LANES = 16

_DIMS_T = (((1,), (1,)), ((), ()))


def _sc_gather_fn():
    info = plsc.get_sparse_core_info()
    nc, ns = info.num_cores, info.num_subcores
    nw = nc * ns
    bpw = B // nw
    nch = bpw // _LANES

    mesh = plsc.VectorSubcoreMesh(core_axis_name="c", subcore_axis_name="s")

    @functools.partial(
        pl.kernel,
        mesh=mesh,
        compiler_params=pltpu.CompilerParams(needs_layout_passes=False),
        out_type=(
            jax.ShapeDtypeStruct((NUM_DEEP_FIELDS, B, 128), jnp.float32),
            jax.ShapeDtypeStruct((B, 1), jnp.float32),
        ),
        scratch_types=[
            pltpu.VMEM((NUM_FIELDS, bpw), jnp.int32),
            pltpu.VMEM((bpw,), jnp.int32),
            pltpu.VMEM((bpw,), jnp.int32),
            pltpu.VMEM((bpw, 128), jnp.float32),
            pltpu.VMEM((bpw, 128), jnp.float32),
            pltpu.VMEM((NUM_FIELDS * VOCAB,), jnp.float32),
            pltpu.VMEM((bpw, 1), jnp.float32),
            pltpu.SemaphoreType.DMA,
            pltpu.SemaphoreType.DMA,
        ],
    )
    def sc_kernel(sparse_hbm, emb_hbm, wide_sp_hbm, e3_hbm, wide_out_hbm,
                  ids_v, idx0_v, idx1_v, rows0_v, rows1_v, wtab_v, wsum_v,
                  gsem, wsem):
        wid = lax.axis_index("s") * nc + lax.axis_index("c")
        base = wid * bpw
        idx_bufs = (idx0_v, idx1_v)
        row_bufs = (rows0_v, rows1_v)

        def build_idx(f):
            buf = idx_bufs[f % 2]
            for c in range(nch):
                buf[pl.ds(c * _LANES, _LANES)] = (
                    ids_v[f, pl.ds(c * _LANES, _LANES)] + f * VOCAB
                )

        pltpu.sync_copy(sparse_hbm.at[:, pl.ds(base, bpw)], ids_v)

        build_idx(0)
        gather = pltpu.async_copy(emb_hbm.at[idx0_v], rows0_v, gsem)

        pltpu.sync_copy(wide_sp_hbm, wtab_v)
        iota = lax.iota(jnp.int32, _LANES)
        zeros = jnp.zeros((_LANES,), jnp.int32)
        for c in range(nch):
            acc = jnp.zeros((_LANES,), jnp.float32)
            for f in range(NUM_FIELDS):
                ids = ids_v[f, pl.ds(c * _LANES, _LANES)] + f * VOCAB
                acc = acc + plsc.load_gather(wtab_v, [ids])
            plsc.store_scatter(wsum_v, [iota + c * _LANES, zeros], acc)
        pltpu.sync_copy(wsum_v, wide_out_hbm.at[pl.ds(base, bpw)])

        write = None
        for f in range(NUM_DEEP_FIELDS):
            cur = f % 2
            if f + 1 < NUM_DEEP_FIELDS:
                build_idx(f + 1)
            gather.wait()
            if write is not None:
                write.wait()
            write = pltpu.async_copy(
                row_bufs[cur], e3_hbm.at[f, pl.ds(base, bpw)], wsem
            )
            if f + 1 < NUM_DEEP_FIELDS:
                gather = pltpu.async_copy(
                    emb_hbm.at[idx_bufs[1 - cur]], row_bufs[1 - cur], gsem
                )
        write.wait()

    return sc_kernel


def _tc_mlp(e3_ref, dense_ref, wsum_ref, dw_ref, db_ref, w1_ref, b1_ref,
            w2_ref, b2_ref, w3_ref, b3_ref, wout_ref, ww13_ref, bias_ref,
            out_ref):
    f32 = jnp.float32
    dot_t = functools.partial(
        lax.dot_general, dimension_numbers=_DIMS_T, preferred_element_type=f32
    )
    dense = dense_ref[...]
    d0 = dot_t(dense, dw_ref[...]) + db_ref[...][None, :]
    hcat = jnp.concatenate(
        [d0] + [e3_ref[f][:, :LATENT] for f in range(NUM_DEEP_FIELDS)], axis=1
    )
    h = jnp.maximum(dot_t(hcat, w1_ref[...]) + b1_ref[...][None, :], 0.0)
    h = jnp.maximum(dot_t(h, w2_ref[...]) + b2_ref[...][None, :], 0.0)
    h = jnp.maximum(dot_t(h, w3_ref[...]) + b3_ref[...][None, :], 0.0)
    deep = jnp.sum(h * wout_ref[...], axis=1, keepdims=True)
    wide_dense = jnp.sum(dense * ww13_ref[...], axis=1, keepdims=True)
    out_ref[...] = deep + wide_dense + wsum_ref[...] + bias_ref[...]


def kernel(sparse_features, dense_features, wide_w, dense_w, dense_b, emb,
           w1, b1, w2, b2, w3, b3, w_out, bias):
    f32 = jnp.float32
    emb_flat = jnp.pad(
        emb.reshape(NUM_DEEP_FIELDS * VOCAB, LATENT),
        ((0, 0), (0, 128 - LATENT)),
    )
    wide_sp = wide_w[NUM_DENSE:]
    e3, wsum = _sc_gather_fn()(sparse_features, emb_flat, wide_sp)

    ww13 = wide_w[:NUM_DENSE][None, :]

    grid = (B // _BT,)
    full = lambda shape: pl.BlockSpec(shape, lambda i: tuple(0 for _ in shape))
    out = pl.pallas_call(
        _tc_mlp,
        grid=grid,
        in_specs=[
            pl.BlockSpec((NUM_DEEP_FIELDS, _BT, 128), lambda i: (0, i, 0)),
            pl.BlockSpec((_BT, NUM_DENSE), lambda i: (i, 0)),
            pl.BlockSpec((_BT, 1), lambda i: (i, 0)),
            full((LATENT, NUM_DENSE)),
            pl.BlockSpec((LATENT,), lambda i: (0,)),
            full((1024, LATENT + D_EMB)),
            pl.BlockSpec((1024,), lambda i: (0,)),
            full((512, 1024)),
            pl.BlockSpec((512,), lambda i: (0,)),
            full((256, 512)),
            pl.BlockSpec((256,), lambda i: (0,)),
            full((1, 256)),
            full((1, NUM_DENSE)),
            full((1, 1)),
        ],
        out_specs=pl.BlockSpec((_BT, 1), lambda i: (i, 0)),
        out_shape=jax.ShapeDtypeStruct((B, 1), f32),
    )(
        e3, dense_features, wsum, dense_w, dense_b, w1, b1, w2, b2, w3, b3,
        w_out, ww13, bias,
    )
    return out


# speedup vs baseline: 1.3832x; 1.1095x over previous
import functools

import jax
import jax.numpy as jnp
from jax import lax
from jax.experimental import pallas as pl
from jax.experimental.pallas import tpu as pltpu
from jax.experimental.pallas import tpu_sc as plsc

B = 4096
NUM_FIELDS = 26
NUM_DEEP_FIELDS = 24
VOCAB = 1000
NUM_DENSE = 13
LATENT = 64
D_EMB = NUM_DEEP_FIELDS * LATENT

_BT = 512
_LANES = 16

_DIMS_T = (((1,), (1,)), ((), ()))


def _sc_gather_fn():
    info = plsc.get_sparse_core_info()
    nc, ns = info.num_cores, info.num_subcores
    nw = nc * ns
    bpw = B // nw
    nch = bpw // _LANES

    mesh = plsc.VectorSubcoreMesh(core_axis_name="c", subcore_axis_name="s")

    @functools.partial(
        pl.kernel,
        mesh=mesh,
        compiler_params=pltpu.CompilerParams(needs_layout_passes=False),
        out_type=(
            jax.ShapeDtypeStruct((NUM_DEEP_FIELDS, B, 128), jnp.float32),
            jax.ShapeDtypeStruct((B, 1), jnp.float32),
        ),
        scratch_types=[
            pltpu.VMEM((NUM_FIELDS, bpw), jnp.int32),
            pltpu.VMEM((4, bpw), jnp.int32),
            pltpu.VMEM((bpw, 128), jnp.float32),
            pltpu.VMEM((bpw, 128), jnp.float32),
            pltpu.VMEM((bpw, 128), jnp.float32),
            pltpu.VMEM((bpw, 128), jnp.float32),
            pltpu.VMEM((NUM_FIELDS * VOCAB,), jnp.float32),
            pltpu.VMEM((bpw, 1), jnp.float32),
            pltpu.SemaphoreType.DMA,
            pltpu.SemaphoreType.DMA,
        ],
    )
    def sc_kernel(sparse_hbm, emb_hbm, wide_sp_hbm, e3_hbm, wide_out_hbm,
                  ids_v, idx_v, rows0_v, rows1_v, rows2_v, rows3_v,
                  wtab_v, wsum_v, gsem, wsem):
        wid = lax.axis_index("s") * nc + lax.axis_index("c")
        base = wid * bpw
        row_bufs = (rows0_v, rows1_v, rows2_v, rows3_v)
        depth = 4
        ahead = 3

        def build_idx(f):
            for c in range(nch):
                idx_v[f % depth, pl.ds(c * _LANES, _LANES)] = (
                    ids_v[f, pl.ds(c * _LANES, _LANES)] + f * VOCAB
                )

        def fire_gather(f):
            return pltpu.async_copy(
                emb_hbm.at[idx_v.at[f % depth]], row_bufs[f % depth], gsem
            )

        pltpu.sync_copy(sparse_hbm.at[:, pl.ds(base, bpw)], ids_v)

        gathers = {}
        writes = {}
        for f in range(ahead):
            build_idx(f)
            gathers[f] = fire_gather(f)

        pltpu.sync_copy(wide_sp_hbm, wtab_v)
        iota = lax.iota(jnp.int32, _LANES)
        zeros = jnp.zeros((_LANES,), jnp.int32)
        for c in range(nch):
            acc = jnp.zeros((_LANES,), jnp.float32)
            for f in range(NUM_FIELDS):
                ids = ids_v[f, pl.ds(c * _LANES, _LANES)] + f * VOCAB
                acc = acc + plsc.load_gather(wtab_v, [ids])
            plsc.store_scatter(wsum_v, [iota + c * _LANES, zeros], acc)
        pltpu.sync_copy(wsum_v, wide_out_hbm.at[pl.ds(base, bpw)])

        for f in range(NUM_DEEP_FIELDS):
            gathers.pop(f).wait()
            writes[f] = pltpu.async_copy(
                row_bufs[f % depth], e3_hbm.at[f, pl.ds(base, bpw)], wsem
            )
            g = f + ahead
            if g < NUM_DEEP_FIELDS:
                build_idx(g)
                if g - depth >= 0:
                    writes.pop(g - depth).wait()
                gathers[g] = fire_gather(g)
        for f in sorted(writes):
            writes.pop(f).wait()

    return sc_kernel


def _tc_mlp(e3_ref, dense_ref, wsum_ref, dw_ref, db_ref, w1_ref, b1_ref,
            w2_ref, b2_ref, w3_ref, b3_ref, wout_ref, ww13_ref, bias_ref,
            out_ref):
    f32 = jnp.float32
    dot_t = functools.partial(
        lax.dot_general, dimension_numbers=_DIMS_T, preferred_element_type=f32
    )
    dense = dense_ref[...]
    d0 = dot_t(dense, dw_ref[...]) + db_ref[...][None, :]
    hcat = jnp.concatenate(
        [d0] + [e3_ref[f][:, :LATENT] for f in range(NUM_DEEP_FIELDS)], axis=1
    )
    h = jnp.maximum(dot_t(hcat, w1_ref[...]) + b1_ref[...][None, :], 0.0)
    h = jnp.maximum(dot_t(h, w2_ref[...]) + b2_ref[...][None, :], 0.0)
    h = jnp.maximum(dot_t(h, w3_ref[...]) + b3_ref[...][None, :], 0.0)
    deep = jnp.sum(h * wout_ref[...], axis=1, keepdims=True)
    wide_dense = jnp.sum(dense * ww13_ref[...], axis=1, keepdims=True)
    out_ref[...] = deep + wide_dense + wsum_ref[...] + bias_ref[...]


def kernel(sparse_features, dense_features, wide_w, dense_w, dense_b, emb,
           w1, b1, w2, b2, w3, b3, w_out, bias):
    f32 = jnp.float32
    emb_flat = jnp.pad(
        emb.reshape(NUM_DEEP_FIELDS * VOCAB, LATENT),
        ((0, 0), (0, 128 - LATENT)),
    )
    wide_sp = wide_w[NUM_DENSE:]
    e3, wsum = _sc_gather_fn()(sparse_features, emb_flat, wide_sp)

    ww13 = wide_w[:NUM_DENSE][None, :]

    grid = (B // _BT,)
    full = lambda shape: pl.BlockSpec(shape, lambda i: tuple(0 for _ in shape))
    out = pl.pallas_call(
        _tc_mlp,
        grid=grid,
        in_specs=[
            pl.BlockSpec((NUM_DEEP_FIELDS, _BT, 128), lambda i: (0, i, 0)),
            pl.BlockSpec((_BT, NUM_DENSE), lambda i: (i, 0)),
            pl.BlockSpec((_BT, 1), lambda i: (i, 0)),
            full((LATENT, NUM_DENSE)),
            pl.BlockSpec((LATENT,), lambda i: (0,)),
            full((1024, LATENT + D_EMB)),
            pl.BlockSpec((1024,), lambda i: (0,)),
            full((512, 1024)),
            pl.BlockSpec((512,), lambda i: (0,)),
            full((256, 512)),
            pl.BlockSpec((256,), lambda i: (0,)),
            full((1, 256)),
            full((1, NUM_DENSE)),
            full((1, 1)),
        ],
        out_specs=pl.BlockSpec((_BT, 1), lambda i: (i, 0)),
        out_shape=jax.ShapeDtypeStruct((B, 1), f32),
    )(
        e3, dense_features, wsum, dense_w, dense_b, w1, b1, w2, b2, w3, b3,
        w_out, ww13, bias,
    )
    return out


# speedup vs baseline: 1.3908x; 1.0055x over previous
import functools

import jax
import jax.numpy as jnp
from jax import lax
from jax.experimental import pallas as pl
from jax.experimental.pallas import tpu as pltpu
from jax.experimental.pallas import tpu_sc as plsc

B = 4096
NUM_FIELDS = 26
NUM_DEEP_FIELDS = 24
VOCAB = 1000
NUM_DENSE = 13
LATENT = 64
D_EMB = NUM_DEEP_FIELDS * LATENT

_BT = 512
_LANES = 16

_DIMS_T = (((1,), (1,)), ((), ()))


def _sc_gather_fn():
    info = plsc.get_sparse_core_info()
    nc, ns = info.num_cores, info.num_subcores
    nw = nc * ns
    bpw = B // nw
    nch = bpw // _LANES

    mesh = plsc.VectorSubcoreMesh(core_axis_name="c", subcore_axis_name="s")

    @functools.partial(
        pl.kernel,
        mesh=mesh,
        compiler_params=pltpu.CompilerParams(needs_layout_passes=False),
        out_type=(
            jax.ShapeDtypeStruct((NUM_DEEP_FIELDS, B, 128), jnp.float32),
            jax.ShapeDtypeStruct((B, 1), jnp.float32),
        ),
        scratch_types=[
            pltpu.VMEM((NUM_FIELDS, bpw), jnp.int32),
            pltpu.VMEM((5, bpw), jnp.int32),
            pltpu.VMEM((bpw, 128), jnp.float32),
            pltpu.VMEM((bpw, 128), jnp.float32),
            pltpu.VMEM((bpw, 128), jnp.float32),
            pltpu.VMEM((bpw, 128), jnp.float32),
            pltpu.VMEM((bpw, 128), jnp.float32),
            pltpu.VMEM((NUM_FIELDS * VOCAB,), jnp.float32),
            pltpu.VMEM((bpw, 1), jnp.float32),
            pltpu.SemaphoreType.DMA,
            pltpu.SemaphoreType.DMA,
        ],
    )
    def sc_kernel(sparse_hbm, emb_hbm, wide_sp_hbm, e3_hbm, wide_out_hbm,
                  ids_v, idx_v, rows0_v, rows1_v, rows2_v, rows3_v,
                  rows4_v, wtab_v, wsum_v, gsem, wsem):
        wid = lax.axis_index("s") * nc + lax.axis_index("c")
        base = wid * bpw
        row_bufs = (rows0_v, rows1_v, rows2_v, rows3_v, rows4_v)
        depth = 5
        ahead = 4

        def build_idx(f):
            for c in range(nch):
                idx_v[f % depth, pl.ds(c * _LANES, _LANES)] = (
                    ids_v[f, pl.ds(c * _LANES, _LANES)] + f * VOCAB
                )

        def fire_gather(f):
            return pltpu.async_copy(
                emb_hbm.at[idx_v.at[f % depth]], row_bufs[f % depth], gsem
            )

        pltpu.sync_copy(sparse_hbm.at[:, pl.ds(base, bpw)], ids_v)

        gathers = {}
        writes = {}
        for f in range(ahead):
            build_idx(f)
            gathers[f] = fire_gather(f)

        pltpu.sync_copy(wide_sp_hbm, wtab_v)
        iota = lax.iota(jnp.int32, _LANES)
        zeros = jnp.zeros((_LANES,), jnp.int32)
        for c in range(nch):
            acc = jnp.zeros((_LANES,), jnp.float32)
            for f in range(NUM_FIELDS):
                ids = ids_v[f, pl.ds(c * _LANES, _LANES)] + f * VOCAB
                acc = acc + plsc.load_gather(wtab_v, [ids])
            plsc.store_scatter(wsum_v, [iota + c * _LANES, zeros], acc)
        pltpu.sync_copy(wsum_v, wide_out_hbm.at[pl.ds(base, bpw)])

        for f in range(NUM_DEEP_FIELDS):
            gathers.pop(f).wait()
            writes[f] = pltpu.async_copy(
                row_bufs[f % depth], e3_hbm.at[f, pl.ds(base, bpw)], wsem
            )
            g = f + ahead
            if g < NUM_DEEP_FIELDS:
                build_idx(g)
                if g - depth >= 0:
                    writes.pop(g - depth).wait()
                gathers[g] = fire_gather(g)
        for f in sorted(writes):
            writes.pop(f).wait()

    return sc_kernel


def _tc_mlp(e3_ref, dense_ref, wsum_ref, dw_ref, db_ref, w1_ref, b1_ref,
            w2_ref, b2_ref, w3_ref, b3_ref, wout_ref, ww13_ref, bias_ref,
            out_ref):
    f32 = jnp.float32
    dot_t = functools.partial(
        lax.dot_general, dimension_numbers=_DIMS_T, preferred_element_type=f32
    )
    dense = dense_ref[...]
    d0 = dot_t(dense, dw_ref[...]) + db_ref[...][None, :]
    hcat = jnp.concatenate(
        [d0] + [e3_ref[f][:, :LATENT] for f in range(NUM_DEEP_FIELDS)], axis=1
    )
    h = jnp.maximum(dot_t(hcat, w1_ref[...]) + b1_ref[...][None, :], 0.0)
    h = jnp.maximum(dot_t(h, w2_ref[...]) + b2_ref[...][None, :], 0.0)
    h = jnp.maximum(dot_t(h, w3_ref[...]) + b3_ref[...][None, :], 0.0)
    deep = jnp.sum(h * wout_ref[...], axis=1, keepdims=True)
    wide_dense = jnp.sum(dense * ww13_ref[...], axis=1, keepdims=True)
    out_ref[...] = deep + wide_dense + wsum_ref[...] + bias_ref[...]


def kernel(sparse_features, dense_features, wide_w, dense_w, dense_b, emb,
           w1, b1, w2, b2, w3, b3, w_out, bias):
    f32 = jnp.float32
    emb_flat = jnp.pad(
        emb.reshape(NUM_DEEP_FIELDS * VOCAB, LATENT),
        ((0, 0), (0, 128 - LATENT)),
    )
    wide_sp = wide_w[NUM_DENSE:]
    e3, wsum = _sc_gather_fn()(sparse_features, emb_flat, wide_sp)

    ww13 = wide_w[:NUM_DENSE][None, :]

    grid = (B // _BT,)
    full = lambda shape: pl.BlockSpec(shape, lambda i: tuple(0 for _ in shape))
    out = pl.pallas_call(
        _tc_mlp,
        grid=grid,
        in_specs=[
            pl.BlockSpec((NUM_DEEP_FIELDS, _BT, 128), lambda i: (0, i, 0)),
            pl.BlockSpec((_BT, NUM_DENSE), lambda i: (i, 0)),
            pl.BlockSpec((_BT, 1), lambda i: (i, 0)),
            full((LATENT, NUM_DENSE)),
            pl.BlockSpec((LATENT,), lambda i: (0,)),
            full((1024, LATENT + D_EMB)),
            pl.BlockSpec((1024,), lambda i: (0,)),
            full((512, 1024)),
            pl.BlockSpec((512,), lambda i: (0,)),
            full((256, 512)),
            pl.BlockSpec((256,), lambda i: (0,)),
            full((1, 256)),
            full((1, NUM_DENSE)),
            full((1, 1)),
        ],
        out_specs=pl.BlockSpec((_BT, 1), lambda i: (i, 0)),
        out_shape=jax.ShapeDtypeStruct((B, 1), f32),
    )(
        e3, dense_features, wsum, dense_w, dense_b, w1, b1, w2, b2, w3, b3,
        w_out, ww13, bias,
    )
    return out
